# Initial kernel scaffold; baseline (speedup 1.0000x reference)
import functools

import jax
import jax.numpy as jnp
from jax import lax
from jax.experimental import pallas as pl
from jax.experimental.pallas import tpu as pltpu
from jax.experimental.pallas import tpu_sc as plsc

_N = 50000
_MP = 25000
_K = 16
_C = 128
_L = 16
_NW = 32
_P = 800
_MP_PAD = _NW * _P
_BP = 8
_NB = _P // _BP
_IDX_ROWS = _MP_PAD * _K // 128


def _reduce_batch(buf, out_v):
    for p in range(_BP):
        r0 = p * _K
        for c in range(_C // _L):
            sl = pl.ds(c * _L, _L)
            acc = buf[r0, sl]
            for j in range(1, _K):
                acc = jnp.maximum(acc, buf[r0 + j, sl])
            out_v[p, sl] = acc


def _pool_body(inp_hbm, idx_hbm, out_hbm,
               idx_v, buf0, buf1, out_v0, out_v1,
               sem_g0, sem_g1, sem_o0, sem_o1):
    wid = lax.axis_index("s") * 2 + lax.axis_index("c")
    idx_base = wid * _NB
    out_base = wid * _P

    pltpu.sync_copy(idx_hbm.at[pl.ds(idx_base, _NB)], idx_v)

    def gather(row, buf, sem):
        pltpu.async_copy(inp_hbm.at[idx_v.at[row]], buf, sem)

    def wait_gather(buf, sem):
        pltpu.make_async_copy(inp_hbm.at[pl.ds(0, _BP * _K)], buf, sem).wait()

    def flush(out_v, row0, sem):
        pltpu.async_copy(out_v, out_hbm.at[pl.ds(row0, _BP)], sem)

    def wait_flush(out_v, sem):
        pltpu.make_async_copy(out_v, out_hbm.at[pl.ds(0, _BP)], sem).wait()

    gather(idx_base, buf0, sem_g0)

    def body(g, _):
        b0 = 2 * g
        b1 = 2 * g + 1
        gather(idx_base + b1, buf1, sem_g1)
        wait_gather(buf0, sem_g0)

        @pl.when(g > 0)
        def _():
            wait_flush(out_v0, sem_o0)

        _reduce_batch(buf0, out_v0)
        flush(out_v0, out_base + b0 * _BP, sem_o0)

        @pl.when(g < _NB // 2 - 1)
        def _():
            gather(idx_base + b0 + 2, buf0, sem_g0)

        wait_gather(buf1, sem_g1)

        @pl.when(g > 0)
        def _():
            wait_flush(out_v1, sem_o1)

        _reduce_batch(buf1, out_v1)
        flush(out_v1, out_base + b1 * _BP, sem_o1)
        return _

    lax.fori_loop(0, _NB // 2, body, None)
    wait_flush(out_v0, sem_o0)
    wait_flush(out_v1, sem_o1)


_pool_call = functools.partial(
    pl.kernel,
    out_type=jax.ShapeDtypeStruct((_MP_PAD, _C), jnp.float32),
    mesh=plsc.VectorSubcoreMesh(core_axis_name="c", subcore_axis_name="s"),
    scratch_types=[
        pltpu.VMEM((_NB, 128), jnp.int32),
        pltpu.VMEM((_BP * _K, _C), jnp.float32),
        pltpu.VMEM((_BP * _K, _C), jnp.float32),
        pltpu.VMEM((_BP, _C), jnp.float32),
        pltpu.VMEM((_BP, _C), jnp.float32),
        pltpu.SemaphoreType.DMA,
        pltpu.SemaphoreType.DMA,
        pltpu.SemaphoreType.DMA,
        pltpu.SemaphoreType.DMA,
    ],
)(_pool_body)


def kernel(inputs, nn_count, nn_index):
    idx = nn_index.astype(jnp.int32)
    cnt = nn_count.astype(jnp.int32)
    valid = jnp.arange(_K, dtype=jnp.int32)[None, :] < cnt[:, None]
    idx = jnp.where(valid, idx, idx[:, :1])
    idx = jnp.pad(idx, ((0, _MP_PAD - _MP), (0, 0)))
    idx = idx.reshape(_IDX_ROWS, 128)
    out = _pool_call(inputs, idx)
    return out[:_MP]


# speedup vs baseline: 1.1956x; 1.1956x over previous
import functools

import jax
import jax.numpy as jnp
from jax import lax
from jax.experimental import pallas as pl
from jax.experimental.pallas import tpu as pltpu
from jax.experimental.pallas import tpu_sc as plsc

_N = 50000
_MP = 25000
_K = 16
_C = 128
_L = 16
_NW = 32
_P = 800
_MP_PAD = _NW * _P
_BP = 8
_NB = _P // _BP


def _reduce_batch(buf, out_v):
    for p in range(_BP):
        r0 = p * _K
        for c in range(_C // _L):
            sl = pl.ds(c * _L, _L)
            acc = buf[r0, sl]
            for j in range(1, _K):
                acc = jnp.maximum(acc, buf[r0 + j, sl])
            out_v[p, sl] = acc


def _pool_body(inp_hbm, idx_hbm, out_hbm,
               idx_v, buf0, buf1, out_v0, out_v1,
               sem_g0, sem_g1, sem_o0, sem_o1):
    wid = lax.axis_index("s") * 2 + lax.axis_index("c")
    out_base = wid * _P

    pltpu.sync_copy(idx_hbm.at[wid], idx_v)

    def gather(row, buf, sem):
        pltpu.async_copy(inp_hbm.at[idx_v.at[row]], buf, sem)

    def wait_gather(buf, sem):
        pltpu.make_async_copy(inp_hbm.at[pl.ds(0, _BP * _K)], buf, sem).wait()

    def flush(out_v, row0, sem):
        row0 = pl.multiple_of(row0, 8)
        pltpu.async_copy(out_v, out_hbm.at[pl.ds(row0, _BP)], sem)

    def wait_flush(out_v, sem):
        pltpu.make_async_copy(out_v, out_hbm.at[pl.ds(0, _BP)], sem).wait()

    gather(0, buf0, sem_g0)

    def body(g, _):
        b0 = 2 * g
        b1 = 2 * g + 1
        gather(b1, buf1, sem_g1)
        wait_gather(buf0, sem_g0)

        @pl.when(g > 0)
        def _():
            wait_flush(out_v0, sem_o0)

        _reduce_batch(buf0, out_v0)
        flush(out_v0, out_base + b0 * _BP, sem_o0)

        @pl.when(g < _NB // 2 - 1)
        def _():
            gather(b0 + 2, buf0, sem_g0)

        wait_gather(buf1, sem_g1)

        @pl.when(g > 0)
        def _():
            wait_flush(out_v1, sem_o1)

        _reduce_batch(buf1, out_v1)
        flush(out_v1, out_base + b1 * _BP, sem_o1)
        return _

    lax.fori_loop(0, _NB // 2, body, None)
    wait_flush(out_v0, sem_o0)
    wait_flush(out_v1, sem_o1)


_pool_call = functools.partial(
    pl.kernel,
    out_type=jax.ShapeDtypeStruct((_MP_PAD, _C), jnp.float32),
    mesh=plsc.VectorSubcoreMesh(core_axis_name="c", subcore_axis_name="s"),
    scratch_types=[
        pltpu.VMEM((_NB, 128), jnp.int32),
        pltpu.VMEM((_BP * _K, _C), jnp.float32),
        pltpu.VMEM((_BP * _K, _C), jnp.float32),
        pltpu.VMEM((_BP, _C), jnp.float32),
        pltpu.VMEM((_BP, _C), jnp.float32),
        pltpu.SemaphoreType.DMA,
        pltpu.SemaphoreType.DMA,
        pltpu.SemaphoreType.DMA,
        pltpu.SemaphoreType.DMA,
    ],
)(_pool_body)


def kernel(inputs, nn_count, nn_index):
    idx = nn_index.astype(jnp.int32)
    cnt = nn_count.astype(jnp.int32)
    valid = jnp.arange(_K, dtype=jnp.int32)[None, :] < cnt[:, None]
    idx = jnp.where(valid, idx, idx[:, :1])
    idx = jnp.pad(idx, ((0, _MP_PAD - _MP), (0, 0)))
    idx = idx.reshape(_NW, _NB, 128)
    out = _pool_call(inputs, idx)
    return out[:_MP]


# speedup vs baseline: 1.2008x; 1.0044x over previous
import functools

import jax
import jax.numpy as jnp
from jax import lax
from jax.experimental import pallas as pl
from jax.experimental.pallas import tpu as pltpu
from jax.experimental.pallas import tpu_sc as plsc

_N = 50000
_MP = 25000
_K = 16
_C = 128
_L = 16
_NW = 32
_P = 800
_MP_PAD = _NW * _P
_BP = 8
_NB = _P // _BP
_NBUF = 4


def _reduce_batch(buf, out_v):
    for p in range(_BP):
        r0 = p * _K
        for c in range(_C // _L):
            sl = pl.ds(c * _L, _L)
            acc = buf[r0, sl]
            for j in range(1, _K):
                acc = jnp.maximum(acc, buf[r0 + j, sl])
            out_v[p, sl] = acc


def _pool_body(inp_hbm, idx_hbm, out_hbm, idx_v, *scratch):
    bufs = scratch[0:_NBUF]
    outs = scratch[_NBUF:2 * _NBUF]
    sem_g = scratch[2 * _NBUF:3 * _NBUF]
    sem_o = scratch[3 * _NBUF:4 * _NBUF]

    wid = lax.axis_index("s") * 2 + lax.axis_index("c")
    out_base = wid * _P

    pltpu.sync_copy(idx_hbm.at[wid], idx_v)

    def gather(row, k):
        pltpu.async_copy(inp_hbm.at[idx_v.at[row]], bufs[k], sem_g[k])

    def wait_gather(k):
        pltpu.make_async_copy(
            inp_hbm.at[pl.ds(0, _BP * _K)], bufs[k], sem_g[k]).wait()

    def flush(k, row0):
        row0 = pl.multiple_of(row0, 8)
        pltpu.async_copy(outs[k], out_hbm.at[pl.ds(row0, _BP)], sem_o[k])

    def wait_flush(k):
        pltpu.make_async_copy(
            outs[k], out_hbm.at[pl.ds(0, _BP)], sem_o[k]).wait()

    for k in range(_NBUF - 1):
        gather(k, k)

    def body(g, _):
        b_first = g * _NBUF
        for k in range(_NBUF):
            b = b_first + k
            nxt = b + _NBUF - 1

            @pl.when(nxt < _NB)
            def _():
                gather(nxt, (k + _NBUF - 1) % _NBUF)

            wait_gather(k)

            @pl.when(g > 0)
            def _():
                wait_flush(k)

            _reduce_batch(bufs[k], outs[k])
            flush(k, out_base + b * _BP)
        return _

    lax.fori_loop(0, _NB // _NBUF, body, None)
    for k in range(_NBUF):
        wait_flush(k)


_pool_call = functools.partial(
    pl.kernel,
    out_type=jax.ShapeDtypeStruct((_MP_PAD, _C), jnp.float32),
    mesh=plsc.VectorSubcoreMesh(core_axis_name="c", subcore_axis_name="s"),
    scratch_types=(
        [pltpu.VMEM((_NB, 128), jnp.int32)]
        + [pltpu.VMEM((_BP * _K, _C), jnp.float32)] * _NBUF
        + [pltpu.VMEM((_BP, _C), jnp.float32)] * _NBUF
        + [pltpu.SemaphoreType.DMA] * (2 * _NBUF)
    ),
)(_pool_body)


def kernel(inputs, nn_count, nn_index):
    idx = nn_index.astype(jnp.int32)
    cnt = nn_count.astype(jnp.int32)
    valid = jnp.arange(_K, dtype=jnp.int32)[None, :] < cnt[:, None]
    idx = jnp.where(valid, idx, idx[:, :1])
    idx = jnp.pad(idx, ((0, _MP_PAD - _MP), (0, 0)))
    idx = idx.reshape(_NW, _NB, 128)
    out = _pool_call(inputs, idx)
    return out[:_MP]


# speedup vs baseline: 3.4774x; 2.8958x over previous
import functools

import jax
import jax.numpy as jnp
from jax import lax
from jax.experimental import pallas as pl
from jax.experimental.pallas import tpu as pltpu
from jax.experimental.pallas import tpu_sc as plsc

_N = 50000
_MP = 25000
_K = 16
_C = 128
_L = 16
_NW = 32
_P = 800
_MP_PAD = _NW * _P
_VMAX = _P * _K
_CH = 128
_RING = 4
_RROWS = _RING * _CH
_OB = 80


def _pool_body(inp_hbm, cnt_hbm, idx_hbm, out_hbm,
               idx_vf, cnt_v, cidx, ring, stage, sems_g, sems_o):
    wid = lax.axis_index("s") * 2 + lax.axis_index("c")
    out_base = wid * _P

    pltpu.sync_copy(idx_hbm.at[pl.ds(wid * _VMAX, _VMAX)], idx_vf)
    pltpu.sync_copy(cnt_hbm.at[pl.ds(wid * _P, _P)], cnt_v.at[pl.ds(0, _P)])

    def compact(p, off):
        row = idx_vf[pl.ds(p * _K, _K)]
        cidx[pl.ds(off, _K)] = row
        cnt = cnt_v[pl.ds(p, _L)][0]
        return off + cnt

    total = lax.fori_loop(0, _P, compact, jnp.int32(0))
    zeros = jnp.zeros((_L,), jnp.int32)
    for t in range(_CH // _L):
        cidx[pl.ds(total + t * _L, _L)] = zeros
    nch = (total + _CH - 1) >> 7

    def gather(chunk):
        slot = chunk & (_RING - 1)
        dst0 = pl.multiple_of(slot * _CH, _CH)
        pltpu.async_copy(
            inp_hbm.at[cidx.at[pl.ds(chunk * _CH, _CH)]],
            ring.at[pl.ds(dst0, _CH)],
            sems_g.at[slot])

    def wait_gather(slot):
        pltpu.make_async_copy(
            inp_hbm.at[pl.ds(0, _CH)], ring.at[pl.ds(0, _CH)],
            sems_g.at[slot]).wait()

    def wait_flush(slot):
        pltpu.make_async_copy(
            stage.at[pl.ds(0, _OB)], out_hbm.at[pl.ds(0, _OB)],
            sems_o.at[slot]).wait()

    for c in range(_RING):
        gather(jnp.int32(c))

    sls = [pl.ds(c * _L, _L) for c in range(_C // _L)]

    def point(p, carry):
        off, gathered, issued = carry
        cnt = cnt_v[pl.ds(p, _L)][0]
        last_chunk = (off + cnt - 1) >> 7

        @pl.when(last_chunk >= gathered)
        def _():
            wait_gather(gathered & (_RING - 1))

        gathered = jnp.where(last_chunk >= gathered, gathered + 1, gathered)

        can_issue = (issued < nch) & ((issued - _RING) < (off >> 7))

        @pl.when(can_issue)
        def _():
            gather(issued)

        issued = jnp.where(can_issue, issued + 1, issued)

        srow = lax.rem(p, jnp.int32(2 * _OB))
        fslot = lax.rem(lax.div(p, jnp.int32(_OB)), jnp.int32(2))

        @pl.when((lax.rem(p, jnp.int32(_OB)) == 0) & (p >= 2 * _OB))
        def _():
            wait_flush(fslot)

        r0 = (off & (_RROWS - 1))
        acc = [ring[r0, sl] for sl in sls]

        def fold(j, acc):
            rr = (off + j) & (_RROWS - 1)
            return tuple(jnp.maximum(a, ring[rr, sl])
                         for a, sl in zip(acc, sls))

        acc = lax.fori_loop(1, cnt, fold, tuple(acc))
        for a, sl in zip(acc, sls):
            stage[srow, sl] = a

        @pl.when(lax.rem(p, jnp.int32(_OB)) == _OB - 1)
        def _():
            s0 = pl.multiple_of(srow - (_OB - 1), _OB)
            d0 = pl.multiple_of(out_base + p - (_OB - 1), 8)
            pltpu.async_copy(stage.at[pl.ds(s0, _OB)],
                             out_hbm.at[pl.ds(d0, _OB)], sems_o.at[fslot])

        return off + cnt, gathered, issued

    lax.fori_loop(0, _P, point,
                  (jnp.int32(0), jnp.int32(0), jnp.int32(_RING)))
    wait_flush(jnp.int32(0))
    wait_flush(jnp.int32(1))


_pool_call = functools.partial(
    pl.kernel,
    out_type=jax.ShapeDtypeStruct((_MP_PAD, _C), jnp.float32),
    mesh=plsc.VectorSubcoreMesh(core_axis_name="c", subcore_axis_name="s"),
    scratch_types=[
        pltpu.VMEM((_VMAX,), jnp.int32),
        pltpu.VMEM((_P + _L,), jnp.int32),
        pltpu.VMEM((_VMAX + _CH + _L,), jnp.int32),
        pltpu.VMEM((_RROWS, _C), jnp.float32),
        pltpu.VMEM((2 * _OB, _C), jnp.float32),
        pltpu.SemaphoreType.DMA((_RING,)),
        pltpu.SemaphoreType.DMA((2,)),
    ],
)(_pool_body)


def kernel(inputs, nn_count, nn_index):
    idx = nn_index.astype(jnp.int32)
    cnt = nn_count.astype(jnp.int32)
    idx = jnp.pad(idx, ((0, _MP_PAD - _MP), (0, 0)))
    cnt = jnp.pad(cnt, (0, _MP_PAD - _MP), constant_values=1)
    out = _pool_call(inputs, cnt, idx.reshape(-1))
    return out[:_MP]


# speedup vs baseline: 4.9441x; 1.4218x over previous
import functools

import jax
import jax.numpy as jnp
from jax import lax
from jax.experimental import pallas as pl
from jax.experimental.pallas import tpu as pltpu
from jax.experimental.pallas import tpu_sc as plsc

_N = 50000
_MP = 25000
_K = 16
_C = 128
_L = 16
_NW = 32
_P = 784
_PCUT = 21
_MP_PAD = _NW * _P
_VMAX = _P * _K + _K
_CH = 128
_RING = 4
_RROWS = _RING * _CH
_OB = 8


def _pool_body(inp_hbm, cnt_hbm, idx_hbm, out_hbm,
               idx_vf, cnt_v, cidx, ring, stage, sems_g, sems_o):
    wid = lax.axis_index("s") * 2 + lax.axis_index("c")
    pw = jnp.where(wid < _PCUT, _P, _P - 8)
    out_base = wid * _P - jnp.maximum(wid - _PCUT, 0) * 8

    pltpu.sync_copy(idx_hbm.at[pl.ds(out_base * _K, _P * _K)],
                    idx_vf.at[pl.ds(0, _P * _K)])
    pltpu.sync_copy(cnt_hbm.at[pl.ds(out_base, _P)], cnt_v.at[pl.ds(0, _P)])

    def compact(p, off):
        row = idx_vf[pl.ds(p * _K, _K)]
        cidx[pl.ds(off, _K)] = row
        cnt = cnt_v[pl.ds(p, _L)][0]
        cidx[pl.ds(off + cnt, _L)] = jnp.full((_L,), row[0], jnp.int32)
        return off + cnt + (cnt & 1)

    total = lax.fori_loop(0, pw, compact, jnp.int32(0))
    zeros = jnp.zeros((_L,), jnp.int32)
    for t in range(_CH // _L):
        cidx[pl.ds(total + t * _L, _L)] = zeros
    nch = (total + _CH - 1) >> 7

    def gather(chunk):
        slot = chunk & (_RING - 1)
        dst0 = pl.multiple_of(slot * _CH, _CH)
        pltpu.async_copy(
            inp_hbm.at[cidx.at[pl.ds(chunk * _CH, _CH)]],
            ring.at[pl.ds(dst0, _CH)],
            sems_g.at[slot])

    def wait_gather(slot):
        pltpu.make_async_copy(
            inp_hbm.at[pl.ds(0, _CH)], ring.at[pl.ds(0, _CH)],
            sems_g.at[slot]).wait()

    def wait_flush(slot):
        pltpu.make_async_copy(
            stage.at[pl.ds(0, _OB)], out_hbm.at[pl.ds(0, _OB)],
            sems_o.at[slot]).wait()

    for c in range(_RING):
        gather(jnp.int32(c))

    sls = [pl.ds(c * _L, _L) for c in range(_C // _L)]

    def point(p, carry):
        off, gathered, issued, srow = carry
        cnt = cnt_v[pl.ds(p, _L)][0]
        cnt_p = cnt + (cnt & 1)
        last_chunk = (off + cnt_p - 1) >> 7

        @pl.when(last_chunk >= gathered)
        def _():
            wait_gather(gathered & (_RING - 1))

        gathered = jnp.where(last_chunk >= gathered, gathered + 1, gathered)

        can_issue = (issued < nch) & ((issued - _RING) < (off >> 7))

        @pl.when(can_issue)
        def _():
            gather(issued)

        issued = jnp.where(can_issue, issued + 1, issued)

        half = srow >> 3

        @pl.when(((srow == 0) | (srow == _OB)) & (p >= 2 * _OB))
        def _():
            wait_flush(half)

        r0 = off & (_RROWS - 1)
        acc = [ring[r0, sl] for sl in sls]

        def fold(j, acc):
            rr = (off + 2 * j) & (_RROWS - 1)
            return tuple(
                jnp.maximum(jnp.maximum(a, ring[rr, sl]), ring[rr + 1, sl])
                for a, sl in zip(acc, sls))

        acc = tuple(jnp.maximum(a, ring[r0 + 1, sl])
                    for a, sl in zip(acc, sls))
        acc = lax.fori_loop(1, cnt_p >> 1, fold, acc)
        for a, sl in zip(acc, sls):
            stage[srow, sl] = a

        @pl.when((srow == _OB - 1) | (srow == 2 * _OB - 1))
        def _():
            s0 = pl.multiple_of((half << 3), _OB)
            d0 = pl.multiple_of(out_base + p - (_OB - 1), 8)
            pltpu.async_copy(stage.at[pl.ds(s0, _OB)],
                             out_hbm.at[pl.ds(d0, _OB)], sems_o.at[half])

        srow = jnp.where(srow == 2 * _OB - 1, 0, srow + 1)
        return off + cnt_p, gathered, issued, srow

    lax.fori_loop(0, pw, point,
                  (jnp.int32(0), jnp.int32(0), jnp.int32(_RING),
                   jnp.int32(0)))
    wait_flush(jnp.int32(0))
    wait_flush(jnp.int32(1))


_pool_call = functools.partial(
    pl.kernel,
    out_type=jax.ShapeDtypeStruct((_MP, _C), jnp.float32),
    mesh=plsc.VectorSubcoreMesh(core_axis_name="c", subcore_axis_name="s"),
    scratch_types=[
        pltpu.VMEM((_P * _K,), jnp.int32),
        pltpu.VMEM((_P + _L,), jnp.int32),
        pltpu.VMEM((_VMAX + _CH + _L,), jnp.int32),
        pltpu.VMEM((_RROWS, _C), jnp.float32),
        pltpu.VMEM((2 * _OB, _C), jnp.float32),
        pltpu.SemaphoreType.DMA((_RING,)),
        pltpu.SemaphoreType.DMA((2,)),
    ],
)(_pool_body)


def kernel(inputs, nn_count, nn_index):
    idx = nn_index.astype(jnp.int32)
    cnt = nn_count.astype(jnp.int32)
    idx = jnp.pad(idx, ((0, _MP_PAD - _MP), (0, 0)))
    cnt = jnp.pad(cnt, (0, _MP_PAD - _MP), constant_values=1)
    return _pool_call(inputs, cnt, idx.reshape(-1))
